# baseline (speedup 1.0000x reference)
import functools

import jax
import jax.numpy as jnp
from jax import lax
from jax.experimental import pallas as pl
from jax.experimental.pallas import tpu as pltpu
from jax.experimental.pallas import tpu_sc as plsc

N = 10000
E = 320000
D = 128
DP = D // 2
DH = D // 4
NCORE = 2
NSUB = 16
EPT = E // NSUB
CHUNK = 80
GCH = 25
NSUPER = EPT // (GCH * CHUNK)
N_PAD = 10240
RPS = N_PAD // NSUB
ZPT = N // NSUB
LANES = 16

_MESH = plsc.VectorSubcoreMesh(core_axis_name="c", subcore_axis_name="s")


@functools.partial(
    pl.kernel,
    mesh=_MESH,
    compiler_params=pltpu.CompilerParams(use_tc_tiling_on_sc=False),
    out_type=jax.ShapeDtypeStruct((NCORE, N_PAD, 2 * DH), jnp.float32),
    scratch_types=[
        pltpu.VMEM((GCH, CHUNK), jnp.int32),
        pltpu.VMEM((GCH, CHUNK), jnp.int32),
        pltpu.VMEM((GCH, CHUNK), jnp.float32),
        pltpu.VMEM((CHUNK, DH), jnp.int32),
        pltpu.VMEM((CHUNK, DH), jnp.int32),
        pltpu.VMEM((CHUNK, 2 * DH), jnp.float32),
        pltpu.VMEM_SHARED((N, DH), jnp.int32),
        pltpu.VMEM_SHARED((N_PAD, 2 * DH), jnp.float32),
        pltpu.SemaphoreType.DMA,
        pltpu.SemaphoreType.DMA,
        pltpu.SemaphoreType.DMA,
    ],
)
def _sc_agg(z_hbm, src_hbm, dst_hbm, a_hbm, out_hbm,
            src_v, dst_v, a_v, pk0, pk1, fbuf, zsh, acc, isem, gsem0, gsem1):
    cid = lax.axis_index("c")
    sid = lax.axis_index("s")
    pk = (pk0, pk1)
    gsem = (gsem0, gsem1)

    pltpu.sync_copy(z_hbm.at[cid, pl.ds(sid * ZPT, ZPT)],
                    zsh.at[pl.ds(sid * ZPT, ZPT)])

    zero16 = jnp.zeros((LANES,), jnp.float32)

    def zrow(r, carry):
        for q in range(2 * DH // LANES):
            fbuf[r, pl.ds(q * LANES, LANES)] = zero16
        return carry

    lax.fori_loop(0, CHUNK, zrow, 0)
    for t in range(RPS // CHUNK):
        pltpu.sync_copy(fbuf, acc.at[pl.ds(sid * RPS + t * CHUNK, CHUNK)])

    plsc.subcore_barrier()

    def scale(j, b):
        for ib in range(CHUNK // LANES):
            av16 = a_v[j, pl.ds(ib * LANES, LANES)]
            for r in range(LANES):
                ab = lax.gather(
                    av16, jnp.full((LANES, 1), r, jnp.int32),
                    dimension_numbers=lax.GatherDimensionNumbers(
                        offset_dims=(), collapsed_slice_dims=(0,),
                        start_index_map=(0,)),
                    slice_sizes=(1,),
                    mode=lax.GatherScatterMode.PROMISE_IN_BOUNDS)
                row = ib * LANES + r
                for w in range(DH // LANES):
                    iv = pk[b][row, pl.ds(w * LANES, LANES)]
                    lo = lax.bitcast_convert_type(
                        lax.shift_left(iv, 16), jnp.float32)
                    hi = lax.bitcast_convert_type(
                        jnp.bitwise_and(iv, jnp.int32(-65536)), jnp.float32)
                    fbuf[row, pl.ds(w * LANES, LANES)] = lo * ab
                    fbuf[row, pl.ds(DH + w * LANES, LANES)] = hi * ab

    def gather_start(j, b):
        pltpu.async_copy(zsh.at[src_v.at[j]], pk[b], gsem[b])

    def gather_wait(j, b):
        pltpu.make_async_copy(zsh.at[src_v.at[j]], pk[b], gsem[b]).wait()

    def scatter(j):
        pltpu.sync_copy(fbuf, acc.at[dst_v.at[j]], add=True)

    def super_body(g, carry):
        c1 = pltpu.async_copy(src_hbm.at[sid, g], src_v, isem)
        c2 = pltpu.async_copy(dst_hbm.at[sid, g], dst_v, isem)
        c3 = pltpu.async_copy(a_hbm.at[sid, g], a_v, isem)
        c1.wait()
        c2.wait()
        c3.wait()

        gather_start(0, 0)

        def pair(q, carry2):
            j = 2 * q
            gather_wait(j, 0)
            gather_start(j + 1, 1)
            scale(j, 0)
            scatter(j)
            gather_wait(j + 1, 1)
            gather_start(j + 2, 0)
            scale(j + 1, 1)
            scatter(j + 1)
            return carry2

        lax.fori_loop(0, (GCH - 1) // 2, pair, 0)
        gather_wait(GCH - 1, 0)
        scale(GCH - 1, 0)
        scatter(GCH - 1)
        return carry

    lax.fori_loop(0, NSUPER, super_body, 0)

    plsc.subcore_barrier()
    pltpu.sync_copy(acc.at[pl.ds(sid * RPS, RPS)],
                    out_hbm.at[cid, pl.ds(sid * RPS, RPS)])


def _ln(h, w, b, eps=1e-5):
    mu = jnp.mean(h, axis=-1, keepdims=True)
    var = jnp.mean((h - mu) * (h - mu), axis=-1, keepdims=True)
    return (h - mu) / jnp.sqrt(var + eps) * w + b


_BR = 1000
_GRID = N // _BR


def _pack_bf16(z):
    zb = z.astype(jnp.bfloat16)
    lo = lax.bitcast_convert_type(zb[:, :DP], jnp.uint16).astype(jnp.int32)
    hi = lax.bitcast_convert_type(zb[:, DP:], jnp.uint16).astype(jnp.int32)
    return jnp.bitwise_or(lo, lax.shift_left(hi, 16))


def _unsplit(p0, p1):
    return jnp.concatenate(
        [p0[:, :DH], p1[:, :DH], p0[:, DH:], p1[:, DH:]], axis=1)


def _tc_in_body(x_ref, w_ref, b_ref, lw_ref, lb_ref, h_ref, z_ref):
    h = jnp.dot(x_ref[...], w_ref[...], preferred_element_type=jnp.float32)
    h = jax.nn.gelu(h + b_ref[...])
    h_ref[...] = h
    z_ref[...] = _pack_bf16(_ln(h, lw_ref[...], lb_ref[...]))


def _tc_mid_body(h_ref, p0_ref, p1_ref, w_ref, b_ref, lw_ref, lb_ref,
                 h1_ref, z1_ref):
    agg = _unsplit(p0_ref[...], p1_ref[...])
    z = jnp.dot(agg, w_ref[...], preferred_element_type=jnp.float32) + b_ref[...]
    h1 = h_ref[...] + jax.nn.gelu(z)
    h1_ref[...] = h1
    z1_ref[...] = _pack_bf16(_ln(h1, lw_ref[...], lb_ref[...]))


def _tc_out_body(h_ref, p0_ref, p1_ref, w_ref, b_ref, lw_ref, lb_ref,
                 wo_ref, bo_ref, out_ref):
    agg = _unsplit(p0_ref[...], p1_ref[...])
    z = jnp.dot(agg, w_ref[...], preferred_element_type=jnp.float32) + b_ref[...]
    h2 = h_ref[...] + jax.nn.gelu(z)
    out_ref[...] = jnp.dot(_ln(h2, lw_ref[...], lb_ref[...]), wo_ref[...],
                           preferred_element_type=jnp.float32) + bo_ref[...]


_ROW_SPEC = pl.BlockSpec((_BR, D), lambda i: (i, 0))
_MAT_SPEC = pl.BlockSpec((D, D), lambda i: (0, 0))
_VEC_SPEC = pl.BlockSpec((1, D), lambda i: (0, 0))
_HALF_SPEC = pl.BlockSpec((_BR, 2 * DH), lambda i: (i, 0))
_PK_SPEC = pl.BlockSpec((_BR, DP), lambda i: (i, 0))
_ND_F32 = jax.ShapeDtypeStruct((N, D), jnp.float32)
_ND_PK = jax.ShapeDtypeStruct((N, DP), jnp.int32)

_tc_in = pl.pallas_call(
    _tc_in_body,
    grid=(_GRID,),
    in_specs=[_ROW_SPEC, _MAT_SPEC, _VEC_SPEC, _VEC_SPEC, _VEC_SPEC],
    out_specs=[_ROW_SPEC, _PK_SPEC],
    out_shape=[_ND_F32, _ND_PK],
)

_tc_mid = pl.pallas_call(
    _tc_mid_body,
    grid=(_GRID,),
    in_specs=[_ROW_SPEC, _HALF_SPEC, _HALF_SPEC, _MAT_SPEC, _VEC_SPEC,
              _VEC_SPEC, _VEC_SPEC],
    out_specs=[_ROW_SPEC, _PK_SPEC],
    out_shape=[_ND_F32, _ND_PK],
)

_tc_out = pl.pallas_call(
    _tc_out_body,
    grid=(_GRID,),
    in_specs=[_ROW_SPEC, _HALF_SPEC, _HALF_SPEC, _MAT_SPEC, _VEC_SPEC,
              _VEC_SPEC, _VEC_SPEC, _MAT_SPEC, _VEC_SPEC],
    out_specs=_ROW_SPEC,
    out_shape=_ND_F32,
)


def kernel(x, edge_index, A_norm, W_in, b_in, ln_w0, ln_b0, W0, b0,
           ln_w1, ln_b1, W1, b1, ln_w_out, ln_b_out, W_out, b_out):
    src3 = edge_index[0].reshape(NSUB, NSUPER, GCH, CHUNK)
    dst3 = edge_index[1].reshape(NSUB, NSUPER, GCH, CHUNK)
    a3 = A_norm.reshape(NSUB, NSUPER, GCH, CHUNK)
    r = lambda v: v.reshape(1, D)

    def split(z):
        return jnp.stack([z[:, :DH], z[:, DH:]], axis=0)

    h, z = _tc_in(x, W_in, r(b_in), r(ln_w0), r(ln_b0))
    p = _sc_agg(split(z), src3, dst3, a3)
    h, z = _tc_mid(h, p[0, :N], p[1, :N], W0, r(b0), r(ln_w1), r(ln_b1))
    p = _sc_agg(split(z), src3, dst3, a3)
    return _tc_out(h, p[0, :N], p[1, :N], W1, r(b1), r(ln_w_out), r(ln_b_out),
                   W_out, r(b_out))
